# Initial kernel scaffold; baseline (speedup 1.0000x reference)
import jax
import jax.numpy as jnp
from jax.experimental import pallas as pl


def kernel(x):
    raise NotImplementedError("write your pallas kernel here")


# speedup vs baseline: 3.0401x; 3.0401x over previous
import functools

import jax
import jax.numpy as jnp
from jax.experimental import pallas as pl
from jax.experimental.pallas import tpu as pltpu


def _bitonic_sort_kernel(x_ref, o_ref, *, n: int):
    x = x_ref[0]
    log_n = n.bit_length() - 1
    idx = jax.lax.broadcasted_iota(jnp.int32, (n, 1), 0)
    zero = jnp.zeros((n, 1), dtype=jnp.int32)
    for k in range(1, log_n + 1):
        asc = (idx & (1 << k)) == zero
        for j in range(k - 1, -1, -1):
            d = 1 << j
            upper = (idx & d) != zero
            keep_min = upper != asc
            fwd = pltpu.roll(x, n - d, 0)
            bwd = pltpu.roll(x, d, 0)
            partner = jnp.where(upper, bwd, fwd)
            x = jnp.where(keep_min, jnp.minimum(x, partner),
                          jnp.maximum(x, partner))
    o_ref[0] = x


@jax.jit
def kernel(x):
    b, n, f = x.shape
    lane_tile = 128
    grid = (b, f // lane_tile)
    return pl.pallas_call(
        functools.partial(_bitonic_sort_kernel, n=n),
        grid=grid,
        in_specs=[pl.BlockSpec((1, n, lane_tile), lambda i, j: (i, 0, j))],
        out_specs=pl.BlockSpec((1, n, lane_tile), lambda i, j: (i, 0, j)),
        out_shape=jax.ShapeDtypeStruct(x.shape, x.dtype),
        compiler_params=pltpu.CompilerParams(
            dimension_semantics=("parallel", "parallel"),
        ),
    )(x)
